# baseline (speedup 1.0000x reference)
import functools

import jax
import jax.numpy as jnp
from jax import lax
from jax.experimental import pallas as pl
from jax.experimental.pallas import tpu as pltpu
from jax.experimental.pallas import tpu_sc as plsc

B, W, L = 1024, 64, 3
DIM = 64
NW = 32
NPAIR = W * L // NW
NBT = B // 128
NCH = NPAIR * NBT

TWO_PI = 6.283185307179586
PI = 3.141592653589793
INV_TWO_PI = 0.15915494309189535
C0 = -0.9999992109801177
C1 = 0.499994213707783
C2 = -0.04165977794574207
C3 = 0.001385879013978696
C4 = -2.420294256311692e-05
C5 = 2.197296441102012e-07

_MESH = plsc.VectorSubcoreMesh(core_axis_name="c", subcore_axis_name="s")


@functools.partial(
    pl.kernel,
    out_type=jax.ShapeDtypeStruct((W, L, 3 * DIM // 8, B // 128, 8, 128),
                                  jnp.float32),
    mesh=_MESH,
    compiler_params=pltpu.CompilerParams(use_tc_tiling_on_sc=False,
                                        needs_layout_passes=False),
    scratch_types=[
        pltpu.VMEM((NPAIR, NBT, 128), jnp.int32),
        pltpu.VMEM((NPAIR, NBT, 128), jnp.int32),
        pltpu.VMEM((NPAIR, NBT, 128), jnp.float32),
        pltpu.VMEM((DIM,), jnp.float32),
        pltpu.VMEM((DIM,), jnp.float32),
        pltpu.VMEM((DIM,), jnp.float32),
        pltpu.VMEM((DIM,), jnp.float32),
        [pltpu.VMEM((128, DIM), jnp.float32) for _ in range(2)],
        [pltpu.VMEM((128, DIM), jnp.float32) for _ in range(2)],
        [pltpu.VMEM((8, 8, 128), jnp.float32) for _ in range(2)],
        [pltpu.VMEM((8, 8, 128), jnp.float32) for _ in range(2)],
        [pltpu.VMEM((8, 8, 128), jnp.float32) for _ in range(2)],
        [pltpu.SemaphoreType.DMA for _ in range(2)],
        [pltpu.SemaphoreType.DMA for _ in range(2)],
        [pltpu.SemaphoreType.DMA for _ in range(2)],
    ],
)
def _cawn_sc(nr_hbm, er_hbm, tr_hbm, ntab_hbm, etab_hbm, fr_hbm, ph_hbm,
             w2_hbm, b2_hbm, out_hbm, nidx, eidx, tbuf,
             frs, phs, w2s, b2s, nrows, erows, ntile, etile, ttile,
             nsem, esem, osem):
    wid = lax.axis_index("s") * 2 + lax.axis_index("c")

    pltpu.sync_copy(fr_hbm, frs)
    pltpu.sync_copy(ph_hbm, phs)
    pltpu.sync_copy(w2_hbm, w2s)
    pltpu.sync_copy(b2_hbm, b2s)

    for pq in range(NPAIR):
        q = wid * NPAIR + pq
        ll, ww = q // W, q % W
        wt, wi = ww // 8, ww % 8
        pltpu.sync_copy(nr_hbm.at[ll, wt, :, wi, :], nidx.at[pq])
        pltpu.sync_copy(er_hbm.at[ll, wt, :, wi, :], eidx.at[pq])
        pltpu.sync_copy(tr_hbm.at[ll, wt, :, wi, :], tbuf.at[pq])

    iota = lax.iota(jnp.int32, 16)

    def issue_gathers(c, b):
        pq, bt = c // NBT, c % NBT
        pltpu.async_copy(ntab_hbm.at[nidx.at[pq, bt]], nrows[b], nsem[b])
        pltpu.async_copy(etab_hbm.at[eidx.at[pq, bt]], erows[b], esem[b])

    def wait_gathers(c, b):
        pq, bt = c // NBT, c % NBT
        pltpu.make_async_copy(ntab_hbm.at[nidx.at[pq, bt]], nrows[b], nsem[b]).wait()
        pltpu.make_async_copy(etab_hbm.at[eidx.at[pq, bt]], erows[b], esem[b]).wait()

    def _out_slices(c):
        pq, bt = c // NBT, c % NBT
        q = wid * NPAIR + pq
        ll, ww = q // W, q % W
        return (out_hbm.at[ww, ll, pl.ds(0, 8), bt],
                out_hbm.at[ww, ll, pl.ds(8, 8), bt],
                out_hbm.at[ww, ll, pl.ds(16, 8), bt])

    def issue_out(c, b):
        sn, se, st = _out_slices(c)
        pltpu.async_copy(ntile[b], sn, osem[b])
        pltpu.async_copy(etile[b], se, osem[b])
        pltpu.async_copy(ttile[b], st, osem[b])

    def drain_out(c, b):
        sn, se, st = _out_slices(c)
        pltpu.make_async_copy(ntile[b], sn, osem[b]).wait()
        pltpu.make_async_copy(etile[b], se, osem[b]).wait()
        pltpu.make_async_copy(ttile[b], st, osem[b]).wait()

    def compute_time(c, b):
        pq, bt = c // NBT, c % NBT
        tdst = ttile[b]

        @plsc.parallel_loop(0, DIM, unroll=2)
        def _f(f):
            fv = jnp.full((16,), f, jnp.int32)
            w = plsc.load_gather(frs, [fv])
            p = plsc.load_gather(phs, [fv])
            w2 = plsc.load_gather(w2s, [fv])
            b2 = plsc.load_gather(b2s, [fv])
            ft, fi = f // 8, f % 8
            for j in range(8):
                tv = tbuf[pq, bt, pl.ds(16 * j, 16)]
                x = tv * w + p
                qq = tv * w2 + b2
                s = x - qq.astype(jnp.int32).astype(jnp.float32) * TWO_PI
                u = s * s
                y = ((((C5 * u + C4) * u + C3) * u + C2) * u + C1) * u + C0
                tdst[ft, fi, pl.ds(16 * j, 16)] = y

    def transpose_tiles(b):
        src_dst = ((nrows[b], ntile[b]), (erows[b], etile[b]))

        @plsc.parallel_loop(0, DIM, unroll=2)
        def _f(f):
            ft, fi = f // 8, f % 8
            fv = jnp.full((16,), f, jnp.int32)
            for src, dst in src_dst:
                vs = [plsc.load_gather(src, [iota + 16 * j, fv])
                      for j in range(8)]
                for j in range(8):
                    dst[ft, fi, pl.ds(16 * j, 16)] = vs[j]

    def phase_step(c, b, first=False, last=False):
        if not last:
            issue_gathers(c + 1, 1 - b)
        compute_time(c, b)
        wait_gathers(c, b)
        transpose_tiles(b)
        if not first:
            drain_out(c - 1, 1 - b)
        issue_out(c, b)

    issue_gathers(0, 0)
    phase_step(0, 0, first=True)
    phase_step(1, 1)

    def pair_steps(cc, carry):
        phase_step(2 * cc, 0)
        phase_step(2 * cc + 1, 1)
        return carry

    lax.fori_loop(1, NCH // 2 - 1, pair_steps, 0)
    phase_step(NCH - 2, 0)
    phase_step(NCH - 1, 1, last=True)
    drain_out(NCH - 1, 1)


def _records_view(x):
    return (x.transpose(2, 1, 0)
            .reshape(L, W // 8, 8, B // 128, 128)
            .transpose(0, 1, 3, 2, 4))


def kernel(node_records, edge_records, t_records, node_table, edge_table,
           basis_freq, phase):
    nr = _records_view(node_records.astype(jnp.int32))
    er = _records_view(edge_records.astype(jnp.int32))
    tr = _records_view(t_records)
    fr = basis_freq
    ph = phase - PI
    w2 = basis_freq * INV_TWO_PI
    b2 = phase * INV_TWO_PI
    out6 = _cawn_sc(nr, er, tr, node_table, edge_table, fr, ph, w2, b2)
    return out6.transpose(3, 5, 0, 1, 2, 4).reshape(B, W, L, 3 * DIM)


# speedup vs baseline: 2.0147x; 2.0147x over previous
import functools

import jax
import jax.numpy as jnp
from jax import lax
from jax.experimental import pallas as pl
from jax.experimental.pallas import tpu as pltpu
from jax.experimental.pallas import tpu_sc as plsc

B, W, L = 1024, 64, 3
DIM = 64
NW = 32
NPAIR = W * L // NW
NBT = B // 128
NCH = NPAIR * NBT

TWO_PI = 6.283185307179586
PI = 3.141592653589793
INV_TWO_PI = 0.15915494309189535
C0 = -0.9999992109801177
C1 = 0.499994213707783
C2 = -0.04165977794574207
C3 = 0.001385879013978696
C4 = -2.420294256311692e-05
C5 = 2.197296441102012e-07

_MESH = plsc.VectorSubcoreMesh(core_axis_name="c", subcore_axis_name="s")


@functools.partial(
    pl.kernel,
    out_type=jax.ShapeDtypeStruct((W, L, 3 * DIM // 8, B // 128, 8, 128),
                                  jnp.float32),
    mesh=_MESH,
    compiler_params=pltpu.CompilerParams(use_tc_tiling_on_sc=False,
                                        needs_layout_passes=False),
    scratch_types=[
        pltpu.VMEM((NPAIR, NBT, 128), jnp.int32),
        pltpu.VMEM((NPAIR, NBT, 128), jnp.int32),
        pltpu.VMEM((NPAIR, NBT, 128), jnp.float32),
        pltpu.VMEM((DIM,), jnp.float32),
        pltpu.VMEM((DIM,), jnp.float32),
        pltpu.VMEM((DIM,), jnp.float32),
        pltpu.VMEM((DIM,), jnp.float32),
        [pltpu.VMEM((128, DIM), jnp.float32) for _ in range(2)],
        [pltpu.VMEM((128, DIM), jnp.float32) for _ in range(2)],
        [pltpu.VMEM((8, 8, 136), jnp.float32) for _ in range(2)],
        [pltpu.VMEM((8, 8, 136), jnp.float32) for _ in range(2)],
        [pltpu.VMEM((8, 8, 136), jnp.float32) for _ in range(2)],
        [pltpu.SemaphoreType.DMA for _ in range(2)],
        [pltpu.SemaphoreType.DMA for _ in range(2)],
        [pltpu.SemaphoreType.DMA for _ in range(2)],
    ],
)
def _cawn_sc(nr_hbm, er_hbm, tr_hbm, ntab_hbm, etab_hbm, fr_hbm, ph_hbm,
             w2_hbm, b2_hbm, out_hbm, nidx, eidx, tbuf,
             frs, phs, w2s, b2s, nrows, erows, ntile, etile, ttile,
             nsem, esem, osem):
    wid = lax.axis_index("s") * 2 + lax.axis_index("c")

    pltpu.sync_copy(fr_hbm, frs)
    pltpu.sync_copy(ph_hbm, phs)
    pltpu.sync_copy(w2_hbm, w2s)
    pltpu.sync_copy(b2_hbm, b2s)

    for pq in range(NPAIR):
        q = wid * NPAIR + pq
        ll, ww = q // W, q % W
        wt, wi = ww // 8, ww % 8
        pltpu.sync_copy(nr_hbm.at[ll, wt, :, wi, :], nidx.at[pq])
        pltpu.sync_copy(er_hbm.at[ll, wt, :, wi, :], eidx.at[pq])
        pltpu.sync_copy(tr_hbm.at[ll, wt, :, wi, :], tbuf.at[pq])

    iota = lax.iota(jnp.int32, 16)

    def issue_gathers(c, b):
        pq, bt = c // NBT, c % NBT
        pltpu.async_copy(ntab_hbm.at[nidx.at[pq, bt]], nrows[b], nsem[b])
        pltpu.async_copy(etab_hbm.at[eidx.at[pq, bt]], erows[b], esem[b])

    def wait_gathers(c, b):
        pq, bt = c // NBT, c % NBT
        pltpu.make_async_copy(ntab_hbm.at[nidx.at[pq, bt]], nrows[b], nsem[b]).wait()
        pltpu.make_async_copy(etab_hbm.at[eidx.at[pq, bt]], erows[b], esem[b]).wait()

    def _out_slices(c):
        pq, bt = c // NBT, c % NBT
        q = wid * NPAIR + pq
        ll, ww = q // W, q % W
        return (out_hbm.at[ww, ll, pl.ds(0, 8), bt],
                out_hbm.at[ww, ll, pl.ds(8, 8), bt],
                out_hbm.at[ww, ll, pl.ds(16, 8), bt])

    def issue_out(c, b):
        sn, se, st = _out_slices(c)
        pltpu.async_copy(ntile[b].at[:, :, pl.ds(0, 128)], sn, osem[b])
        pltpu.async_copy(etile[b].at[:, :, pl.ds(0, 128)], se, osem[b])
        pltpu.async_copy(ttile[b].at[:, :, pl.ds(0, 128)], st, osem[b])

    def drain_out(c, b):
        sn, se, st = _out_slices(c)
        pltpu.make_async_copy(ntile[b].at[:, :, pl.ds(0, 128)], sn, osem[b]).wait()
        pltpu.make_async_copy(etile[b].at[:, :, pl.ds(0, 128)], se, osem[b]).wait()
        pltpu.make_async_copy(ttile[b].at[:, :, pl.ds(0, 128)], st, osem[b]).wait()

    def compute_time(c, b):
        pq, bt = c // NBT, c % NBT
        tdst = ttile[b]

        @plsc.parallel_loop(0, DIM, unroll=2)
        def _f(f):
            fv = jnp.full((16,), f, jnp.int32)
            w = plsc.load_gather(frs, [fv])
            p = plsc.load_gather(phs, [fv])
            w2 = plsc.load_gather(w2s, [fv])
            b2 = plsc.load_gather(b2s, [fv])
            ft, fi = f // 8, f % 8
            for j in range(8):
                tv = tbuf[pq, bt, pl.ds(16 * j, 16)]
                x = tv * w + p
                qq = tv * w2 + b2
                s = x - qq.astype(jnp.int32).astype(jnp.float32) * TWO_PI
                u = s * s
                y = ((((C5 * u + C4) * u + C3) * u + C2) * u + C1) * u + C0
                tdst[ft, fi, pl.ds(16 * j, 16)] = y

    d0s = [(iota + 16 * fk) // 8 for fk in range(4)]
    d1s = [(iota + 16 * fk) % 8 for fk in range(4)]

    def transpose_tiles(b):
        src_dst = ((nrows[b], ntile[b]), (erows[b], etile[b]))

        @plsc.parallel_loop(0, 128, unroll=2)
        def _b(bb):
            bv = jnp.full((16,), bb, jnp.int32)
            for src, dst in src_dst:
                for fk in range(4):
                    v = src[bb, pl.ds(16 * fk, 16)]
                    plsc.store_scatter(dst, [d0s[fk], d1s[fk], bv], v)

    def phase_step(c, b, first=False, last=False):
        if not last:
            issue_gathers(c + 1, 1 - b)
        compute_time(c, b)
        wait_gathers(c, b)
        transpose_tiles(b)
        if not first:
            drain_out(c - 1, 1 - b)
        issue_out(c, b)

    issue_gathers(0, 0)
    phase_step(0, 0, first=True)
    phase_step(1, 1)

    def pair_steps(cc, carry):
        phase_step(2 * cc, 0)
        phase_step(2 * cc + 1, 1)
        return carry

    lax.fori_loop(1, NCH // 2 - 1, pair_steps, 0)
    phase_step(NCH - 2, 0)
    phase_step(NCH - 1, 1, last=True)
    drain_out(NCH - 1, 1)


def _records_view(x):
    return (x.transpose(2, 1, 0)
            .reshape(L, W // 8, 8, B // 128, 128)
            .transpose(0, 1, 3, 2, 4))


def kernel(node_records, edge_records, t_records, node_table, edge_table,
           basis_freq, phase):
    nr = _records_view(node_records.astype(jnp.int32))
    er = _records_view(edge_records.astype(jnp.int32))
    tr = _records_view(t_records)
    fr = basis_freq
    ph = phase - PI
    w2 = basis_freq * INV_TWO_PI
    b2 = phase * INV_TWO_PI
    out6 = _cawn_sc(nr, er, tr, node_table, edge_table, fr, ph, w2, b2)
    return out6.transpose(3, 5, 0, 1, 2, 4).reshape(B, W, L, 3 * DIM)


# speedup vs baseline: 2.0617x; 1.0233x over previous
import functools

import jax
import jax.numpy as jnp
from jax import lax
from jax.experimental import pallas as pl
from jax.experimental.pallas import tpu as pltpu
from jax.experimental.pallas import tpu_sc as plsc

B, W, L = 1024, 64, 3
DIM = 64
NW = 32
NPAIR = W * L // NW
NBT = B // 128
NCH = NPAIR * NBT

TWO_PI = 6.283185307179586
PI = 3.141592653589793
INV_TWO_PI = 0.15915494309189535
C0 = -0.9999992109801177
C1 = 0.499994213707783
C2 = -0.04165977794574207
C3 = 0.001385879013978696
C4 = -2.420294256311692e-05
C5 = 2.197296441102012e-07

_MESH = plsc.VectorSubcoreMesh(core_axis_name="c", subcore_axis_name="s")


@functools.partial(
    pl.kernel,
    out_type=jax.ShapeDtypeStruct((W, L, 3 * DIM // 8, B // 128, 8, 128),
                                  jnp.float32),
    mesh=_MESH,
    compiler_params=pltpu.CompilerParams(use_tc_tiling_on_sc=False,
                                        needs_layout_passes=False),
    scratch_types=[
        pltpu.VMEM((NPAIR, NBT, 128), jnp.int32),
        pltpu.VMEM((NPAIR, NBT, 128), jnp.int32),
        pltpu.VMEM((NPAIR, NBT, 128), jnp.float32),
        pltpu.VMEM((DIM,), jnp.float32),
        pltpu.VMEM((DIM,), jnp.float32),
        pltpu.VMEM((DIM,), jnp.float32),
        pltpu.VMEM((DIM,), jnp.float32),
        [pltpu.VMEM((128, DIM), jnp.float32) for _ in range(3)],
        [pltpu.VMEM((128, DIM), jnp.float32) for _ in range(3)],
        [pltpu.VMEM((8, 8, 136), jnp.float32) for _ in range(2)],
        [pltpu.VMEM((8, 8, 136), jnp.float32) for _ in range(2)],
        [pltpu.VMEM((8, 8, 136), jnp.float32) for _ in range(2)],
        [pltpu.SemaphoreType.DMA for _ in range(3)],
        [pltpu.SemaphoreType.DMA for _ in range(3)],
        [pltpu.SemaphoreType.DMA for _ in range(2)],
    ],
)
def _cawn_sc(nr_hbm, er_hbm, tr_hbm, ntab_hbm, etab_hbm, fr_hbm, ph_hbm,
             w2_hbm, b2_hbm, out_hbm, nidx, eidx, tbuf,
             frs, phs, w2s, b2s, nrows, erows, ntile, etile, ttile,
             nsem, esem, osem):
    wid = lax.axis_index("s") * 2 + lax.axis_index("c")

    prologue = [(fr_hbm, frs), (ph_hbm, phs), (w2_hbm, w2s), (b2_hbm, b2s)]
    for pq in range(NPAIR):
        q = wid * NPAIR + pq
        ll, ww = q // W, q % W
        wt, wi = ww // 8, ww % 8
        prologue += [(nr_hbm.at[ll, wt, :, wi, :], nidx.at[pq]),
                     (er_hbm.at[ll, wt, :, wi, :], eidx.at[pq]),
                     (tr_hbm.at[ll, wt, :, wi, :], tbuf.at[pq])]
    for src, dst in prologue:
        pltpu.async_copy(src, dst, osem[0])
    for src, dst in prologue:
        pltpu.make_async_copy(src, dst, osem[0]).wait()

    iota = lax.iota(jnp.int32, 16)

    def issue_gathers(c, b):
        pq, bt = c // NBT, c % NBT
        pltpu.async_copy(ntab_hbm.at[nidx.at[pq, bt]], nrows[b], nsem[b])
        pltpu.async_copy(etab_hbm.at[eidx.at[pq, bt]], erows[b], esem[b])

    def wait_gathers(c, b):
        pq, bt = c // NBT, c % NBT
        pltpu.make_async_copy(ntab_hbm.at[nidx.at[pq, bt]], nrows[b], nsem[b]).wait()
        pltpu.make_async_copy(etab_hbm.at[eidx.at[pq, bt]], erows[b], esem[b]).wait()

    def _out_slices(c):
        pq, bt = c // NBT, c % NBT
        q = wid * NPAIR + pq
        ll, ww = q // W, q % W
        return (out_hbm.at[ww, ll, pl.ds(0, 8), bt],
                out_hbm.at[ww, ll, pl.ds(8, 8), bt],
                out_hbm.at[ww, ll, pl.ds(16, 8), bt])

    def issue_out(c, b):
        sn, se, st = _out_slices(c)
        pltpu.async_copy(ntile[b].at[:, :, pl.ds(0, 128)], sn, osem[b])
        pltpu.async_copy(etile[b].at[:, :, pl.ds(0, 128)], se, osem[b])
        pltpu.async_copy(ttile[b].at[:, :, pl.ds(0, 128)], st, osem[b])

    def drain_out(c, b):
        sn, se, st = _out_slices(c)
        pltpu.make_async_copy(ntile[b].at[:, :, pl.ds(0, 128)], sn, osem[b]).wait()
        pltpu.make_async_copy(etile[b].at[:, :, pl.ds(0, 128)], se, osem[b]).wait()
        pltpu.make_async_copy(ttile[b].at[:, :, pl.ds(0, 128)], st, osem[b]).wait()

    def compute_time(c, b):
        pq, bt = c // NBT, c % NBT
        tdst = ttile[b]

        @plsc.parallel_loop(0, DIM, unroll=2)
        def _f(f):
            fv = jnp.full((16,), f, jnp.int32)
            w = plsc.load_gather(frs, [fv])
            p = plsc.load_gather(phs, [fv])
            w2 = plsc.load_gather(w2s, [fv])
            b2 = plsc.load_gather(b2s, [fv])
            ft, fi = f // 8, f % 8
            for j in range(8):
                tv = tbuf[pq, bt, pl.ds(16 * j, 16)]
                x = tv * w + p
                qq = tv * w2 + b2
                s = x - qq.astype(jnp.int32).astype(jnp.float32) * TWO_PI
                u = s * s
                y = ((((C5 * u + C4) * u + C3) * u + C2) * u + C1) * u + C0
                tdst[ft, fi, pl.ds(16 * j, 16)] = y

    d0s = [(iota + 16 * fk) // 8 for fk in range(4)]
    d1s = [(iota + 16 * fk) % 8 for fk in range(4)]

    def transpose_tiles(r, b):
        src_dst = ((nrows[r], ntile[b]), (erows[r], etile[b]))

        @plsc.parallel_loop(0, 128, unroll=2)
        def _b(bb):
            bv = jnp.full((16,), bb, jnp.int32)
            for src, dst in src_dst:
                for fk in range(4):
                    v = src[bb, pl.ds(16 * fk, 16)]
                    plsc.store_scatter(dst, [d0s[fk], d1s[fk], bv], v)

    def phase_step(c, r, b):
        @pl.when(c >= 2)
        def _():
            drain_out(c - 2, b)

        compute_time(c, b)
        wait_gathers(c, r)
        transpose_tiles(r, b)

        @pl.when(c + 2 < NCH)
        def _():
            issue_gathers(c + 2, (r + 2) % 3)

        issue_out(c, b)

    issue_gathers(0, 0)
    issue_gathers(1, 1)

    def block_steps(cc, carry):
        for k in range(6):
            phase_step(6 * cc + k, k % 3, k % 2)
        return carry

    lax.fori_loop(0, NCH // 6, block_steps, 0)
    drain_out(NCH - 2, 0)
    drain_out(NCH - 1, 1)


def _records_view(x):
    return (x.transpose(2, 1, 0)
            .reshape(L, W // 8, 8, B // 128, 128)
            .transpose(0, 1, 3, 2, 4))


def kernel(node_records, edge_records, t_records, node_table, edge_table,
           basis_freq, phase):
    nr = _records_view(node_records.astype(jnp.int32))
    er = _records_view(edge_records.astype(jnp.int32))
    tr = _records_view(t_records)
    fr = basis_freq
    ph = phase - PI
    w2 = basis_freq * INV_TWO_PI
    b2 = phase * INV_TWO_PI
    out6 = _cawn_sc(nr, er, tr, node_table, edge_table, fr, ph, w2, b2)
    return out6.transpose(3, 5, 0, 1, 2, 4).reshape(B, W, L, 3 * DIM)


# speedup vs baseline: 2.2504x; 1.0915x over previous
import functools

import jax
import jax.numpy as jnp
from jax import lax
from jax.experimental import pallas as pl
from jax.experimental.pallas import tpu as pltpu
from jax.experimental.pallas import tpu_sc as plsc

B, W, L = 1024, 64, 3
DIM = 64
NW = 32
NPAIR = W * L // NW
CHUNK = 64
NCH = NPAIR * B // CHUNK // 2 * 2
assert NCH == NPAIR * B // CHUNK

TWO_PI = 6.283185307179586
PI = 3.141592653589793
INV_TWO_PI = 0.15915494309189535
C0 = -0.9999992109801177
C1 = 0.499994213707783
C2 = -0.04165977794574207
C3 = 0.001385879013978696
C4 = -2.420294256311692e-05
C5 = 2.197296441102012e-07

_MESH = plsc.VectorSubcoreMesh(core_axis_name="c", subcore_axis_name="s")


@functools.partial(
    pl.kernel,
    out_type=jax.ShapeDtypeStruct((W, L, 3 * DIM // 8, B // 128, 8, 128),
                                  jnp.float32),
    mesh=_MESH,
    compiler_params=pltpu.CompilerParams(use_tc_tiling_on_sc=False,
                                        needs_layout_passes=False),
    scratch_types=[
        pltpu.VMEM((NPAIR, 8, 128), jnp.int32),
        pltpu.VMEM((NPAIR, 8, 128), jnp.int32),
        pltpu.VMEM((NPAIR, 8, 128), jnp.float32),
        pltpu.VMEM((DIM,), jnp.float32),
        pltpu.VMEM((DIM,), jnp.float32),
        [pltpu.VMEM((CHUNK, 128), jnp.float32) for _ in range(3)],
        [pltpu.VMEM((CHUNK, 128), jnp.float32) for _ in range(3)],
        [pltpu.VMEM((8, 8, CHUNK + 8), jnp.float32) for _ in range(2)],
        [pltpu.VMEM((8, 8, CHUNK + 8), jnp.float32) for _ in range(2)],
        [pltpu.VMEM((8, 8, CHUNK), jnp.float32) for _ in range(2)],
        [pltpu.SemaphoreType.DMA for _ in range(3)],
        [pltpu.SemaphoreType.DMA for _ in range(3)],
        [pltpu.SemaphoreType.DMA for _ in range(2)],
    ],
)
def _cawn_sc(nr_hbm, er_hbm, tr_hbm, tab_hbm, fr_hbm, ph_hbm,
             out_hbm, nidx, eidx, tbuf, frs, phs, nrows, erows,
             ntile, etile, ttile, nsem, esem, osem):
    wid = lax.axis_index("s") * 2 + lax.axis_index("c")

    prologue = [(fr_hbm, frs), (ph_hbm, phs)]
    for pq in range(NPAIR):
        q = wid * NPAIR + pq
        ll, ww = q // W, q % W
        wt, wi = ww // 8, ww % 8
        prologue += [(nr_hbm.at[ll, wt, :, wi, :], nidx.at[pq]),
                     (er_hbm.at[ll, wt, :, wi, :], eidx.at[pq]),
                     (tr_hbm.at[ll, wt, :, wi, :], tbuf.at[pq])]
    for src, dst in prologue:
        pltpu.async_copy(src, dst, osem[0])
    for src, dst in prologue:
        pltpu.make_async_copy(src, dst, osem[0]).wait()

    iota = lax.iota(jnp.int32, 16)

    def _chunk_addr(c):
        return c // 16, (c // 2) % 8, c % 2

    def issue_gathers(c, r):
        pq, bt, h = _chunk_addr(c)
        pltpu.async_copy(tab_hbm.at[nidx.at[pq, bt, pl.ds(CHUNK * h, CHUNK)]],
                         nrows[r], nsem[r])
        pltpu.async_copy(tab_hbm.at[eidx.at[pq, bt, pl.ds(CHUNK * h, CHUNK)]],
                         erows[r], esem[r])

    def wait_gathers(c, r):
        pq, bt, h = _chunk_addr(c)
        pltpu.make_async_copy(
            tab_hbm.at[nidx.at[pq, bt, pl.ds(CHUNK * h, CHUNK)]],
            nrows[r], nsem[r]).wait()
        pltpu.make_async_copy(
            tab_hbm.at[eidx.at[pq, bt, pl.ds(CHUNK * h, CHUNK)]],
            erows[r], esem[r]).wait()

    def _out_slices(c):
        pq, bt, h = _chunk_addr(c)
        q = wid * NPAIR + pq
        ll, ww = q // W, q % W
        bsl = pl.ds(CHUNK * h, CHUNK)
        return (out_hbm.at[ww, ll, pl.ds(0, 8), bt, :, bsl],
                out_hbm.at[ww, ll, pl.ds(8, 8), bt, :, bsl],
                out_hbm.at[ww, ll, pl.ds(16, 8), bt, :, bsl])

    def issue_out(c, b):
        sn, se, st = _out_slices(c)
        pltpu.async_copy(ntile[b].at[:, :, pl.ds(0, CHUNK)], sn, osem[b])
        pltpu.async_copy(etile[b].at[:, :, pl.ds(0, CHUNK)], se, osem[b])
        pltpu.async_copy(ttile[b], st, osem[b])

    def drain_out(c, b):
        sn, se, st = _out_slices(c)
        pltpu.make_async_copy(ntile[b].at[:, :, pl.ds(0, CHUNK)], sn, osem[b]).wait()
        pltpu.make_async_copy(etile[b].at[:, :, pl.ds(0, CHUNK)], se, osem[b]).wait()
        pltpu.make_async_copy(ttile[b], st, osem[b]).wait()

    def compute_time(c, b):
        pq, bt, h = _chunk_addr(c)
        tdst = ttile[b]

        @plsc.parallel_loop(0, DIM, unroll=2)
        def _f(f):
            fv = jnp.full((16,), f, jnp.int32)
            w = plsc.load_gather(frs, [fv])
            p = plsc.load_gather(phs, [fv])
            ft, fi = f // 8, f % 8
            for j in range(CHUNK // 16):
                tv = tbuf[pq, bt, pl.ds(CHUNK * h + 16 * j, 16)]
                x = tv * w + p
                qq = x * INV_TWO_PI + 0.5
                s = x - qq.astype(jnp.int32).astype(jnp.float32) * TWO_PI
                u = s * s
                y = ((((C5 * u + C4) * u + C3) * u + C2) * u + C1) * u + C0
                tdst[ft, fi, pl.ds(16 * j, 16)] = y

    d0s = [(iota + 16 * fk) // 8 for fk in range(4)]
    d1s = [(iota + 16 * fk) % 8 for fk in range(4)]

    def transpose_tiles(r, b):
        src_dst = ((nrows[r], 0, ntile[b]), (erows[r], DIM, etile[b]))

        @plsc.parallel_loop(0, CHUNK, unroll=2)
        def _b(bb):
            bv = jnp.full((16,), bb, jnp.int32)
            for src, off, dst in src_dst:
                for fk in range(4):
                    v = src[bb, pl.ds(off + 16 * fk, 16)]
                    plsc.store_scatter(dst, [d0s[fk], d1s[fk], bv], v)

    def phase_step(c, r, b):
        @pl.when(c >= 2)
        def _():
            drain_out(c - 2, b)

        compute_time(c, b)
        wait_gathers(c, r)
        transpose_tiles(r, b)

        @pl.when(c + 2 < NCH)
        def _():
            issue_gathers(c + 2, (r + 2) % 3)

        issue_out(c, b)

    issue_gathers(0, 0)
    issue_gathers(1, 1)

    def block_steps(cc, carry):
        for k in range(6):
            phase_step(6 * cc + k, k % 3, k % 2)
        return carry

    lax.fori_loop(0, NCH // 6, block_steps, 0)
    drain_out(NCH - 2, 0)
    drain_out(NCH - 1, 1)


def _records_view(x):
    return (x.transpose(2, 1, 0)
            .reshape(L, W // 8, 8, B // 128, 128)
            .transpose(0, 1, 3, 2, 4))


def kernel(node_records, edge_records, t_records, node_table, edge_table,
           basis_freq, phase):
    nr = _records_view(node_records.astype(jnp.int32))
    er = _records_view(edge_records.astype(jnp.int32))
    tr = _records_view(t_records)
    tab = jnp.concatenate([node_table, edge_table], axis=1)
    fr = basis_freq
    ph = phase - PI
    out6 = _cawn_sc(nr, er, tr, tab, fr, ph)
    return out6.transpose(3, 5, 0, 1, 2, 4).reshape(B, W, L, 3 * DIM)
